# baseline (speedup 1.0000x reference)
import jax
import jax.numpy as jnp
from jax.experimental import pallas as pl
from jax.experimental.pallas import tpu as pltpu

_TB = 64
_IN_BUFS = 10


def _outer(x_hbm, gv_hbm, g_hbm, we_hbm, be_hbm, wd_hbm, bd_hbm, out_hbm,
           g_v, we_v, be_v, wd_v, bd_v, sem):
    size = x_hbm.shape[1]
    nsteps = x_hbm.shape[0] // _TB

    copies = [
        pltpu.make_async_copy(we_hbm, we_v, sem),
        pltpu.make_async_copy(wd_hbm, wd_v, sem),
        pltpu.make_async_copy(g_hbm, g_v, sem),
        pltpu.make_async_copy(be_hbm, be_v, sem),
        pltpu.make_async_copy(bd_hbm, bd_v, sem),
    ]
    for c in copies:
        c.start()

    def inner(idx, x_blk, gv_blk, out_blk):
        @pl.when(idx[0] == 0)
        def _wait_params():
            for c in copies:
                c.wait()

        xa = x_blk[...] + g_v[...] * gv_blk[...]
        h = jnp.dot(xa, we_v[...], preferred_element_type=jnp.float32)
        h = jnp.maximum(h + be_v[...], 0.0)
        out = jnp.dot(h, wd_v[...], preferred_element_type=jnp.float32)
        out_blk[...] = out + bd_v[...]

    pltpu.emit_pipeline(
        inner,
        grid=(nsteps,),
        in_specs=[
            pl.BlockSpec((_TB, size), lambda i: (i, 0),
                         pipeline_mode=pl.Buffered(buffer_count=_IN_BUFS)),
            pl.BlockSpec((_TB, size), lambda i: (i, 0),
                         pipeline_mode=pl.Buffered(buffer_count=_IN_BUFS)),
        ],
        out_specs=[
            pl.BlockSpec((_TB, size), lambda i: (i, 0),
                         pipeline_mode=pl.Buffered(buffer_count=2)),
        ],
        _explicit_indices=True,
    )(x_hbm, gv_hbm, out_hbm)


def kernel(x, genre_vec, g, W_enc, b_enc, W_dec, b_dec):
    B, SIZE = x.shape
    HIDDEN = W_enc.shape[1]
    b_enc2 = b_enc.reshape(1, HIDDEN)
    b_dec2 = b_dec.reshape(1, SIZE)

    return pl.pallas_call(
        _outer,
        in_specs=[pl.BlockSpec(memory_space=pl.ANY)] * 7,
        out_specs=pl.BlockSpec(memory_space=pl.ANY),
        out_shape=jax.ShapeDtypeStruct((B, SIZE), jnp.float32),
        scratch_shapes=[
            pltpu.VMEM((1, SIZE), jnp.float32),
            pltpu.VMEM((SIZE, HIDDEN), jnp.float32),
            pltpu.VMEM((1, HIDDEN), jnp.float32),
            pltpu.VMEM((HIDDEN, SIZE), jnp.float32),
            pltpu.VMEM((1, SIZE), jnp.float32),
            pltpu.SemaphoreType.DMA,
        ],
    )(x, genre_vec, g, W_enc, b_enc2, W_dec, b_dec2)


# speedup vs baseline: 1.0003x; 1.0003x over previous
import jax
import jax.numpy as jnp
from jax.experimental import pallas as pl
from jax.experimental.pallas import tpu as pltpu

_TB = 64
_IN_BUFS = 6


def _outer(x_hbm, gv_hbm, g_hbm, we_hbm, be_hbm, wd_hbm, bd_hbm, out_hbm,
           g_v, we_v, be_v, wd_v, bd_v, sem):
    size = x_hbm.shape[1]
    nsteps = x_hbm.shape[0] // _TB

    copies = [
        pltpu.make_async_copy(we_hbm, we_v, sem),
        pltpu.make_async_copy(wd_hbm, wd_v, sem),
        pltpu.make_async_copy(g_hbm, g_v, sem),
        pltpu.make_async_copy(be_hbm, be_v, sem),
        pltpu.make_async_copy(bd_hbm, bd_v, sem),
    ]
    for c in copies:
        c.start()

    def inner(idx, x_blk, gv_blk, out_blk):
        @pl.when(idx[0] == 0)
        def _wait_params():
            for c in copies:
                c.wait()

        xa = x_blk[...] + g_v[...] * gv_blk[...]
        h = jnp.dot(xa, we_v[...], preferred_element_type=jnp.float32)
        h = jnp.maximum(h + be_v[...], 0.0)
        out = jnp.dot(h, wd_v[...], preferred_element_type=jnp.float32)
        out_blk[...] = out + bd_v[...]

    pltpu.emit_pipeline(
        inner,
        grid=(nsteps,),
        in_specs=[
            pl.BlockSpec((_TB, size), lambda i: (i, 0),
                         pipeline_mode=pl.Buffered(buffer_count=_IN_BUFS)),
            pl.BlockSpec((_TB, size), lambda i: (i, 0),
                         pipeline_mode=pl.Buffered(buffer_count=_IN_BUFS)),
        ],
        out_specs=[
            pl.BlockSpec((_TB, size), lambda i: (i, 0),
                         pipeline_mode=pl.Buffered(buffer_count=2)),
        ],
        _explicit_indices=True,
    )(x_hbm, gv_hbm, out_hbm)


def kernel(x, genre_vec, g, W_enc, b_enc, W_dec, b_dec):
    B, SIZE = x.shape
    HIDDEN = W_enc.shape[1]
    b_enc2 = b_enc.reshape(1, HIDDEN)
    b_dec2 = b_dec.reshape(1, SIZE)

    return pl.pallas_call(
        _outer,
        in_specs=[pl.BlockSpec(memory_space=pl.ANY)] * 7,
        out_specs=pl.BlockSpec(memory_space=pl.ANY),
        out_shape=jax.ShapeDtypeStruct((B, SIZE), jnp.float32),
        scratch_shapes=[
            pltpu.VMEM((1, SIZE), jnp.float32),
            pltpu.VMEM((SIZE, HIDDEN), jnp.float32),
            pltpu.VMEM((1, HIDDEN), jnp.float32),
            pltpu.VMEM((HIDDEN, SIZE), jnp.float32),
            pltpu.VMEM((1, SIZE), jnp.float32),
            pltpu.SemaphoreType.DMA,
        ],
    )(x, genre_vec, g, W_enc, b_enc2, W_dec, b_dec2)
